# baseline (speedup 1.0000x reference)
import functools
import math

import jax
import jax.numpy as jnp
from jax import lax
from jax.experimental import pallas as pl
from jax.experimental.pallas import tpu as pltpu
from jax.experimental.pallas import tpu_sc as plsc

D_MODEL = 64
SCALE = math.sqrt(D_MODEL)
NC, NS, L = 2, 16, 16
NW = NC * NS
CHUNK = 128
RING = 4


@functools.partial(jax.jit, static_argnames=("n_rows",))
def _sc_embed(x2d, table, n_rows):
    nch = x2d.shape[0] // NW
    b_per_w = nch * CHUNK
    mesh = plsc.VectorSubcoreMesh(
        core_axis_name="c", subcore_axis_name="s", num_cores=NC, num_subcores=NS
    )

    @functools.partial(
        pl.kernel,
        out_type=jax.ShapeDtypeStruct((n_rows, D_MODEL), jnp.float32),
        mesh=mesh,
        scratch_types=[
            pltpu.VMEM((nch, CHUNK), jnp.int32),
            pltpu.VMEM((RING, CHUNK, D_MODEL), jnp.float32),
            pltpu.SemaphoreType.DMA((RING,)),
            pltpu.SemaphoreType.DMA((RING,)),
        ],
        compiler_params=pltpu.CompilerParams(use_tc_tiling_on_sc=False),
    )
    def k(x_hbm, tab_hbm, out_hbm, idx_v, rows_v, gsem, wsem):
        wid = lax.axis_index("s") * NC + lax.axis_index("c")
        chunk0 = wid * nch
        row0 = wid * b_per_w

        pltpu.sync_copy(x_hbm.at[pl.ds(chunk0, nch)], idx_v)

        def start_gather(j, b):
            pltpu.async_copy(tab_hbm.at[idx_v.at[j]], rows_v.at[b], gsem.at[b])

        def wait_gather(j, b):
            pltpu.make_async_copy(
                tab_hbm.at[idx_v.at[j]], rows_v.at[b], gsem.at[b]
            ).wait()

        def out_slice(j):
            return out_hbm.at[pl.ds(row0 + j * CHUNK, CHUNK)]

        def start_write(j, b):
            pltpu.async_copy(rows_v.at[b], out_slice(j), wsem.at[b])

        def wait_write(j, b):
            pltpu.make_async_copy(rows_v.at[b], out_slice(j), wsem.at[b]).wait()

        for b in range(RING - 1):
            start_gather(b, b)

        @pl.loop(0, nch // RING)
        def _grp(g):
            for b in range(RING):
                j = g * RING + b
                wait_gather(j, b)
                buf = rows_v.at[b]

                @pl.loop(0, CHUNK, unroll=4)
                def _row(i):
                    for kk in range(D_MODEL // L):
                        sl = pl.ds(kk * L, L)
                        buf[i, sl] = buf[i, sl] * SCALE

                start_write(j, b)
                jn = j + RING - 1
                bn = (b + RING - 1) % RING

                @pl.when(jn < nch)
                def _():
                    @pl.when(jn >= RING)
                    def _():
                        wait_write(jn - RING, bn)

                    start_gather(jn, bn)

        for b in range(RING):
            wait_write(nch - RING + b, b)

    return k(x2d, table)


def kernel(x, table):
    n_rows = x.shape[0] * x.shape[1]
    x2d = x.reshape(n_rows // CHUNK, CHUNK).astype(jnp.int32)
    out = _sc_embed(x2d, table, n_rows)
    return out.reshape(*x.shape, D_MODEL)
